# Initial kernel scaffold; baseline (speedup 1.0000x reference)
import jax
import jax.numpy as jnp
from jax.experimental import pallas as pl


def kernel(x, c1w, c1b, c2w, c2b, c3w, c3b, w1h, b1h, wv2p, bv2p, wa2p, ba2p, mask, S):
    raise NotImplementedError("write your pallas kernel here")


# speedup vs baseline: 84.1817x; 84.1817x over previous
import functools

import jax
import jax.numpy as jnp
import numpy as np
from jax.experimental import pallas as pl
from jax.experimental.pallas import tpu as pltpu

_ATOM_PAD = 128
_HID = 512
_NB = 16
_TB = 256


def _conv_stack_kernel(xs_ref, w1_ref, b1_ref, w2_ref, b2_ref, w3_ref, b3_ref,
                       feat_ref, p1_ref, p2_ref, p3_ref):
    nb = _NB
    xs = xs_ref[...]
    for a in range(2):
        for b in range(2):
            t = a * 2 + b
            p1_ref[:, :, :, t * 64:(t + 1) * 64] = xs[:, a:a + 20, b:b + 20, :]
    y1 = jnp.dot(p1_ref[...].reshape(nb * 400, 256), w1_ref[...],
                 preferred_element_type=jnp.float32)
    y1 = jnp.maximum(y1 + b1_ref[...], 0.0).astype(jnp.bfloat16)
    c1 = y1.reshape(nb, 20, 20, 128)

    c1r = c1.reshape(nb, 10, 2, 10, 2, 128)
    for i in range(4):
        for j in range(4):
            t = i * 4 + j
            val = c1r[:, (i >> 1):(i >> 1) + 9, i & 1,
                      (j >> 1):(j >> 1) + 9, j & 1, :32]
            p2_ref[:, :, 0:9, t * 32:(t + 1) * 32] = val
    y2 = jnp.dot(p2_ref[...].reshape(nb * 144, 512), w2_ref[...],
                 preferred_element_type=jnp.float32)
    y2 = jnp.maximum(y2 + b2_ref[...], 0.0).astype(jnp.bfloat16)
    c2 = y2.reshape(nb, 9, 16, 128)

    for i in range(3):
        for j in range(3):
            t = i * 3 + j
            p3_ref[:, :, 0:7, t * 64:(t + 1) * 64] = c2[:, i:i + 7, j:j + 7, :64]
    y3 = jnp.dot(p3_ref[...].reshape(nb * 112, 576), w3_ref[...],
                 preferred_element_type=jnp.float32)
    y3 = jnp.maximum(y3 + b3_ref[...], 0.0).astype(jnp.bfloat16)
    c3 = y3.reshape(nb, 7, 16, 128)

    for p in range(7):
        for q in range(7):
            feat_ref[:, (p * 7 + q) * 64:(p * 7 + q + 1) * 64] = c3[:, p, q, :64]


def _head_kernel(f_ref, w1_ref, b1_ref, wv2_ref, bv2_ref, wa2_ref, ba2_ref,
                 mask_ref, sup_ref, q_ref, dist_ref, *, n_actions):
    h = jnp.dot(f_ref[...], w1_ref[...], preferred_element_type=jnp.float32)
    h = jnp.maximum(h + b1_ref[...], 0.0)
    hv = h[:, :_HID]
    ha = h[:, _HID:]

    value = jnp.dot(hv, wv2_ref[...], preferred_element_type=jnp.float32) + bv2_ref[...]
    adv = jnp.dot(ha, wa2_ref[...], preferred_element_type=jnp.float32) + ba2_ref[...]

    adv_mean = adv[:, :_ATOM_PAD]
    for a in range(1, n_actions):
        adv_mean = adv_mean + adv[:, a * _ATOM_PAD:(a + 1) * _ATOM_PAD]
    adv_mean = adv_mean * (1.0 / n_actions)

    base = value - adv_mean + mask_ref[...]
    for a in range(n_actions):
        qa = base + adv[:, a * _ATOM_PAD:(a + 1) * _ATOM_PAD]
        m = jnp.max(qa, axis=-1, keepdims=True)
        e = jnp.exp(qa - m)
        s = jnp.sum(e, axis=-1, keepdims=True)
        inv = pl.reciprocal(s, approx=True)
        dist_ref[:, a * _ATOM_PAD:(a + 1) * _ATOM_PAD] = jnp.maximum(e * inv, 0.001)

    q_ref[...] = jnp.dot(dist_ref[...], sup_ref[...],
                         preferred_element_type=jnp.float32)


def _conv1_row_perm():
    perm = np.empty(256, np.int32)
    for a in range(2):
        for b in range(2):
            for di in range(4):
                for dj in range(4):
                    for c in range(4):
                        new = (a * 2 + b) * 64 + di * 16 + dj * 4 + c
                        old = (4 * a + di) * 32 + (4 * b + dj) * 4 + c
                        perm[new] = old
    return perm


def kernel(x, c1w, c1b, c2w, c2b, c3w, c3b, w1h, b1h, wv2p, bv2p, wa2p, ba2p,
           mask, S):
    n_actions = 18
    B = x.shape[0]

    xs = x.reshape(B, 4, 21, 4, 21, 4).transpose(0, 2, 4, 3, 5, 1)
    xs = xs.reshape(B, 21, 21, 64).astype(jnp.bfloat16)
    w1p = c1w[_conv1_row_perm(), :]

    nsteps = B // _NB
    feature = pl.pallas_call(
        _conv_stack_kernel,
        out_shape=jax.ShapeDtypeStruct((B, 3136), jnp.bfloat16),
        grid=(nsteps,),
        in_specs=[
            pl.BlockSpec((_NB, 21, 21, 64), lambda i: (i, 0, 0, 0)),
            pl.BlockSpec((256, 128), lambda i: (0, 0)),
            pl.BlockSpec((1, 128), lambda i: (0, 0)),
            pl.BlockSpec((512, 128), lambda i: (0, 0)),
            pl.BlockSpec((1, 128), lambda i: (0, 0)),
            pl.BlockSpec((576, 128), lambda i: (0, 0)),
            pl.BlockSpec((1, 128), lambda i: (0, 0)),
        ],
        out_specs=pl.BlockSpec((_NB, 3136), lambda i: (i, 0)),
        scratch_shapes=[
            pltpu.VMEM((_NB, 20, 20, 256), jnp.bfloat16),
            pltpu.VMEM((_NB, 9, 16, 512), jnp.bfloat16),
            pltpu.VMEM((_NB, 7, 16, 576), jnp.bfloat16),
        ],
        compiler_params=pltpu.CompilerParams(
            dimension_semantics=("parallel",),
            vmem_limit_bytes=100 * 1024 * 1024,
        ),
    )(xs, w1p, c1b.reshape(1, 128).astype(jnp.float32),
      c2w, c2b.reshape(1, 128).astype(jnp.float32),
      c3w, c3b.reshape(1, 128).astype(jnp.float32))

    tb = min(_TB, B)
    q = pl.pallas_call(
        functools.partial(_head_kernel, n_actions=n_actions),
        out_shape=jax.ShapeDtypeStruct((B, _ATOM_PAD), jnp.float32),
        grid=(B // tb,),
        in_specs=[
            pl.BlockSpec((tb, 3136), lambda i: (i, 0)),
            pl.BlockSpec((3136, 2 * _HID), lambda i: (0, 0)),
            pl.BlockSpec((1, 2 * _HID), lambda i: (0, 0)),
            pl.BlockSpec((_HID, _ATOM_PAD), lambda i: (0, 0)),
            pl.BlockSpec((1, _ATOM_PAD), lambda i: (0, 0)),
            pl.BlockSpec((_HID, n_actions * _ATOM_PAD), lambda i: (0, 0)),
            pl.BlockSpec((1, n_actions * _ATOM_PAD), lambda i: (0, 0)),
            pl.BlockSpec((1, _ATOM_PAD), lambda i: (0, 0)),
            pl.BlockSpec((n_actions * _ATOM_PAD, _ATOM_PAD), lambda i: (0, 0)),
        ],
        out_specs=pl.BlockSpec((tb, _ATOM_PAD), lambda i: (i, 0)),
        scratch_shapes=[pltpu.VMEM((tb, n_actions * _ATOM_PAD), jnp.float32)],
        compiler_params=pltpu.CompilerParams(
            dimension_semantics=("parallel",),
            vmem_limit_bytes=100 * 1024 * 1024,
        ),
    )(feature, w1h, b1h, wv2p, bv2p, wa2p, ba2p, mask, S)
    return q[:, :n_actions]
